# Initial kernel scaffold; baseline (speedup 1.0000x reference)
import jax
import jax.numpy as jnp
from jax.experimental import pallas as pl


def kernel(x, edge_index, batch, W1, b1, W2, b2, linW, linb):
    raise NotImplementedError("write your pallas kernel here")


# speedup vs baseline: 13.0970x; 13.0970x over previous
import functools

import jax
import jax.numpy as jnp
from jax import lax
from jax.experimental import pallas as pl
from jax.experimental.pallas import tpu as pltpu
from jax.experimental.pallas import tpu_sc as plsc

N = 10000
E = 320000
F = 128
G = 16
NC = 2
NS = 16
NW = NC * NS
EP = E // NW
K = 80
NCH = EP // K
RPT = 624
TAIL = N - NS * RPT
RB = 2000
NBLK = N // RB

_mesh = plsc.VectorSubcoreMesh(
    core_axis_name="c", subcore_axis_name="s", num_cores=NC, num_subcores=NS)


@functools.partial(
    pl.kernel,
    out_type=jax.ShapeDtypeStruct((NC, N, 8), jnp.float32),
    mesh=_mesh,
    scratch_types=[
        pltpu.VMEM((K,), jnp.int32),
        pltpu.VMEM((K, 8), jnp.float32),
        pltpu.VMEM_SHARED((N, 8), jnp.float32),
    ],
)
def _sc_degree(dst_hbm, ones_hbm, out_hbm, didx, onesrow, sp_deg):
    c = lax.axis_index("c")
    s = lax.axis_index("s")
    pltpu.sync_copy(ones_hbm.at[pl.ds(s * RPT, RPT)],
                    sp_deg.at[pl.ds(s * RPT, RPT)])

    @pl.when(s == NS - 1)
    def _init_tail():
        pltpu.sync_copy(ones_hbm.at[pl.ds(NS * RPT, TAIL)],
                        sp_deg.at[pl.ds(NS * RPT, TAIL)])

    pltpu.sync_copy(ones_hbm.at[pl.ds(0, K)], onesrow)
    plsc.subcore_barrier()
    base = (c * (E // NC) + s * EP)
    def body(j, carry):
        off = base + j * K
        pltpu.sync_copy(dst_hbm.at[pl.ds(off, K)], didx)
        pltpu.sync_copy(onesrow, sp_deg.at[didx], add=True)
        return carry
    lax.fori_loop(0, NCH, body, 0)
    plsc.subcore_barrier()
    pltpu.sync_copy(sp_deg.at[pl.ds(s * RPT, RPT)],
                    out_hbm.at[c, pl.ds(s * RPT, RPT)])

    @pl.when(s == NS - 1)
    def _out_tail():
        pltpu.sync_copy(sp_deg.at[pl.ds(NS * RPT, TAIL)],
                        out_hbm.at[c, pl.ds(NS * RPT, TAIL)])


@functools.partial(
    pl.kernel,
    out_type=jax.ShapeDtypeStruct((NC, N, F), jnp.float32),
    mesh=_mesh,
    scratch_types=[
        pltpu.VMEM((K,), jnp.int32),
        pltpu.VMEM((K,), jnp.int32),
        pltpu.VMEM((K, F), jnp.float32),
        pltpu.VMEM_SHARED((N, F), jnp.float32),
        pltpu.SemaphoreType.DMA,
    ],
)
def _sc_aggregate(hs_hbm, src_hbm, dst_hbm, out_hbm, sidx, didx, rows,
                  sp_agg, sem):
    c = lax.axis_index("c")
    s = lax.axis_index("s")
    pltpu.sync_copy(hs_hbm.at[pl.ds(s * RPT, RPT)],
                    sp_agg.at[pl.ds(s * RPT, RPT)])

    @pl.when(s == NS - 1)
    def _init_tail():
        pltpu.sync_copy(hs_hbm.at[pl.ds(NS * RPT, TAIL)],
                        sp_agg.at[pl.ds(NS * RPT, TAIL)])

    plsc.subcore_barrier()
    base = (c * (E // NC) + s * EP)
    def body(j, carry):
        off = base + j * K
        pltpu.sync_copy(src_hbm.at[pl.ds(off, K)], sidx)
        pltpu.sync_copy(dst_hbm.at[pl.ds(off, K)], didx)
        pltpu.async_copy(hs_hbm.at[sidx], rows, sem).wait()
        pltpu.sync_copy(rows, sp_agg.at[didx], add=True)
        return carry
    lax.fori_loop(0, NCH, body, 0)
    plsc.subcore_barrier()
    pltpu.sync_copy(sp_agg.at[pl.ds(s * RPT, RPT)],
                    out_hbm.at[c, pl.ds(s * RPT, RPT)])

    @pl.when(s == NS - 1)
    def _out_tail():
        pltpu.sync_copy(sp_agg.at[pl.ds(NS * RPT, TAIL)],
                        out_hbm.at[c, pl.ds(NS * RPT, TAIL)])


def _dinv_block(d0, d1):
    return lax.rsqrt(d0[:, :1] + d1[:, :1] - 1.0)


def _tc_first_body(x_ref, w_ref, d0_ref, d1_ref, o_ref):
    dv = _dinv_block(d0_ref[...], d1_ref[...])
    h = jnp.dot(x_ref[...], w_ref[...], preferred_element_type=jnp.float32)
    o_ref[...] = h * dv


def _tc_first(x, W1, degf):
    return pl.pallas_call(
        _tc_first_body,
        grid=(NBLK,),
        in_specs=[
            pl.BlockSpec((RB, F), lambda i: (i, 0)),
            pl.BlockSpec((F, F), lambda i: (0, 0)),
            pl.BlockSpec((RB, 8), lambda i: (i, 0)),
            pl.BlockSpec((RB, 8), lambda i: (i + NBLK, 0)),
        ],
        out_specs=pl.BlockSpec((RB, F), lambda i: (i, 0)),
        out_shape=jax.ShapeDtypeStruct((N, F), jnp.float32),
    )(x, W1, degf, degf)


def _tc_mid_body(a0_ref, a1_ref, hs_ref, d0_ref, d1_ref, b_ref, w_ref, o_ref):
    dv = _dinv_block(d0_ref[...], d1_ref[...])
    hs = hs_ref[...]
    z = jax.nn.relu(dv * (a0_ref[...] + a1_ref[...] - hs) + b_ref[...])
    o_ref[...] = jnp.dot(z, w_ref[...], preferred_element_type=jnp.float32) * dv


def _tc_mid(aggf, hs, degf, b, W):
    return pl.pallas_call(
        _tc_mid_body,
        grid=(NBLK,),
        in_specs=[
            pl.BlockSpec((RB, F), lambda i: (i, 0)),
            pl.BlockSpec((RB, F), lambda i: (i + NBLK, 0)),
            pl.BlockSpec((RB, F), lambda i: (i, 0)),
            pl.BlockSpec((RB, 8), lambda i: (i, 0)),
            pl.BlockSpec((RB, 8), lambda i: (i + NBLK, 0)),
            pl.BlockSpec((1, F), lambda i: (0, 0)),
            pl.BlockSpec((F, F), lambda i: (0, 0)),
        ],
        out_specs=pl.BlockSpec((RB, F), lambda i: (i, 0)),
        out_shape=jax.ShapeDtypeStruct((N, F), jnp.float32),
    )(aggf, aggf, hs, degf, degf, b, W)


def _tc_final_body(a0_ref, a1_ref, hs_ref, d0_ref, d1_ref, b_ref, bt_ref,
                   lw_ref, lb_ref, o_ref, acc, cnt):
    i = pl.program_id(0)

    @pl.when(i == 0)
    def _init():
        acc[...] = jnp.zeros_like(acc)
        cnt[...] = jnp.zeros_like(cnt)

    dv = _dinv_block(d0_ref[...], d1_ref[...])
    hs = hs_ref[...]
    z = jax.nn.relu(dv * (a0_ref[...] + a1_ref[...] - hs) + b_ref[...])
    bt = bt_ref[...].reshape(1, RB)
    oh = (lax.broadcasted_iota(jnp.int32, (G, RB), 0) == bt).astype(jnp.float32)
    acc[...] += jnp.dot(oh, z, preferred_element_type=jnp.float32)
    cnt[...] += jnp.broadcast_to(jnp.sum(oh, axis=1, keepdims=True), (G, F))

    @pl.when(i == NBLK - 1)
    def _fin():
        pooled = acc[...] / jnp.maximum(cnt[...], 1.0)
        o_ref[...] = jnp.dot(pooled, lw_ref[...],
                             preferred_element_type=jnp.float32) + lb_ref[...]


def _tc_final(aggf, hs, degf, b, batch2d, linW, linb2d):
    C = linW.shape[1]
    return pl.pallas_call(
        _tc_final_body,
        grid=(NBLK,),
        in_specs=[
            pl.BlockSpec((RB, F), lambda i: (i, 0)),
            pl.BlockSpec((RB, F), lambda i: (i + NBLK, 0)),
            pl.BlockSpec((RB, F), lambda i: (i, 0)),
            pl.BlockSpec((RB, 8), lambda i: (i, 0)),
            pl.BlockSpec((RB, 8), lambda i: (i + NBLK, 0)),
            pl.BlockSpec((1, F), lambda i: (0, 0)),
            pl.BlockSpec((RB, 1), lambda i: (i, 0)),
            pl.BlockSpec((F, C), lambda i: (0, 0)),
            pl.BlockSpec((1, C), lambda i: (0, 0)),
        ],
        out_specs=pl.BlockSpec((G, C), lambda i: (0, 0)),
        out_shape=jax.ShapeDtypeStruct((G, C), jnp.float32),
        scratch_shapes=[
            pltpu.VMEM((G, F), jnp.float32),
            pltpu.VMEM((G, F), jnp.float32),
        ],
    )(aggf, aggf, hs, degf, degf, b, batch2d, linW, linb2d)


def kernel(x, edge_index, batch, W1, b1, W2, b2, linW, linb):
    src = edge_index[0]
    dst = edge_index[1]
    ones = jnp.ones((N, 8), jnp.float32)

    degf = _sc_degree(dst, ones).reshape(NC * N, 8)
    hs1 = _tc_first(x, W1, degf)
    agg1 = _sc_aggregate(hs1, src, dst).reshape(NC * N, F)
    hs2 = _tc_mid(agg1, hs1, degf, b1.reshape(1, F), W2)
    agg2 = _sc_aggregate(hs2, src, dst).reshape(NC * N, F)
    return _tc_final(agg2, hs2, degf, b2.reshape(1, F),
                     batch.reshape(N, 1), linW, linb.reshape(1, linb.shape[0]))


# speedup vs baseline: 27.4180x; 2.0935x over previous
import functools

import jax
import jax.numpy as jnp
from jax import lax
from jax.experimental import pallas as pl
from jax.experimental.pallas import tpu as pltpu
from jax.experimental.pallas import tpu_sc as plsc

N = 10000
E = 320000
F = 128
G = 16
NC = 2
NS = 16
NW = NC * NS
EP = E // NW
K = 128
NCH = EP // K
KT = EP - NCH * K
RPT = 624
TAIL = N - NS * RPT
RB = 2000
NBLK = N // RB

_mesh = plsc.VectorSubcoreMesh(
    core_axis_name="c", subcore_axis_name="s", num_cores=NC, num_subcores=NS)


@functools.partial(
    pl.kernel,
    out_type=jax.ShapeDtypeStruct((NC, N, 8), jnp.float32),
    mesh=_mesh,
    scratch_types=[
        pltpu.VMEM((2, K), jnp.int32),
        pltpu.VMEM((K, 8), jnp.float32),
        pltpu.VMEM((KT,), jnp.int32),
        pltpu.VMEM_SHARED((N, 8), jnp.float32),
        pltpu.SemaphoreType.DMA,
        pltpu.SemaphoreType.DMA,
    ],
)
def _sc_degree(dst_hbm, ones_hbm, out_hbm, didx, onesrow, tdidx, sp_deg,
               si0, si1):
    c = lax.axis_index("c")
    s = lax.axis_index("s")
    pltpu.sync_copy(ones_hbm.at[pl.ds(s * RPT, RPT)],
                    sp_deg.at[pl.ds(s * RPT, RPT)])

    @pl.when(s == NS - 1)
    def _init_tail():
        pltpu.sync_copy(ones_hbm.at[pl.ds(NS * RPT, TAIL)],
                        sp_deg.at[pl.ds(NS * RPT, TAIL)])

    pltpu.sync_copy(ones_hbm.at[pl.ds(0, K)], onesrow)
    plsc.subcore_barrier()
    base = (c * (E // NC) + s * EP)
    sems = (si0, si1)
    for b in range(2):
        pltpu.async_copy(dst_hbm.at[pl.ds(base + b * K, K)], didx.at[b],
                         sems[b])

    def body2(g, carry):
        for b in range(2):
            j = 2 * g + b
            pltpu.make_async_copy(dst_hbm.at[pl.ds(base, K)], didx.at[b],
                                  sems[b]).wait()
            pltpu.sync_copy(onesrow, sp_deg.at[didx.at[b]], add=True)

            @pl.when(j + 2 < NCH)
            def _next():
                pltpu.async_copy(dst_hbm.at[pl.ds(base + (j + 2) * K, K)],
                                 didx.at[b], sems[b])
        return carry

    lax.fori_loop(0, NCH // 2, body2, 0)
    pltpu.sync_copy(dst_hbm.at[pl.ds(base + NCH * K, KT)], tdidx)
    pltpu.sync_copy(onesrow.at[pl.ds(0, KT)], sp_deg.at[tdidx], add=True)
    plsc.subcore_barrier()
    pltpu.sync_copy(sp_deg.at[pl.ds(s * RPT, RPT)],
                    out_hbm.at[c, pl.ds(s * RPT, RPT)])

    @pl.when(s == NS - 1)
    def _out_tail():
        pltpu.sync_copy(sp_deg.at[pl.ds(NS * RPT, TAIL)],
                        out_hbm.at[c, pl.ds(NS * RPT, TAIL)])


@functools.partial(
    pl.kernel,
    out_type=jax.ShapeDtypeStruct((NC, N, F), jnp.float32),
    mesh=_mesh,
    scratch_types=[
        pltpu.VMEM((2, K), jnp.int32),
        pltpu.VMEM((2, K), jnp.int32),
        pltpu.VMEM((2, K, F), jnp.float32),
        pltpu.VMEM((KT,), jnp.int32),
        pltpu.VMEM((KT,), jnp.int32),
        pltpu.VMEM((KT, F), jnp.float32),
        pltpu.VMEM_SHARED((N, F), jnp.float32),
        pltpu.SemaphoreType.DMA,
        pltpu.SemaphoreType.DMA,
        pltpu.SemaphoreType.DMA,
        pltpu.SemaphoreType.DMA,
        pltpu.SemaphoreType.DMA,
        pltpu.SemaphoreType.DMA,
    ],
)
def _sc_aggregate(hs_hbm, src_hbm, dst_hbm, out_hbm, sidx, didx, rows,
                  tsidx, tdidx, trows, sp_agg, ssi0, ssi1, sdi0, sdi1,
                  sg0, sg1):
    c = lax.axis_index("c")
    s = lax.axis_index("s")
    pltpu.sync_copy(hs_hbm.at[pl.ds(s * RPT, RPT)],
                    sp_agg.at[pl.ds(s * RPT, RPT)])

    @pl.when(s == NS - 1)
    def _init_tail():
        pltpu.sync_copy(hs_hbm.at[pl.ds(NS * RPT, TAIL)],
                        sp_agg.at[pl.ds(NS * RPT, TAIL)])

    plsc.subcore_barrier()
    base = (c * (E // NC) + s * EP)
    ssi = (ssi0, ssi1)
    sdi = (sdi0, sdi1)
    sg = (sg0, sg1)
    for b in range(2):
        pltpu.async_copy(src_hbm.at[pl.ds(base + b * K, K)], sidx.at[b],
                         ssi[b])
        pltpu.async_copy(dst_hbm.at[pl.ds(base + b * K, K)], didx.at[b],
                         sdi[b])
    pltpu.make_async_copy(src_hbm.at[pl.ds(base, K)], sidx.at[0],
                          ssi[0]).wait()
    pltpu.async_copy(hs_hbm.at[sidx.at[0]], rows.at[0], sg[0])

    def body2(g, carry):
        for b in range(2):
            j = 2 * g + b
            ob = 1 - b
            pltpu.make_async_copy(hs_hbm.at[sidx.at[b]], rows.at[b],
                                  sg[b]).wait()

            @pl.when(j + 1 < NCH)
            def _issue_next_gather():
                pltpu.make_async_copy(src_hbm.at[pl.ds(base, K)],
                                      sidx.at[ob], ssi[ob]).wait()
                pltpu.async_copy(hs_hbm.at[sidx.at[ob]], rows.at[ob], sg[ob])

            pltpu.make_async_copy(dst_hbm.at[pl.ds(base, K)], didx.at[b],
                                  sdi[b]).wait()
            pltpu.sync_copy(rows.at[b], sp_agg.at[didx.at[b]], add=True)

            @pl.when(j + 2 < NCH)
            def _stage_next_idx():
                off = base + (j + 2) * K
                pltpu.async_copy(src_hbm.at[pl.ds(off, K)], sidx.at[b],
                                 ssi[b])
                pltpu.async_copy(dst_hbm.at[pl.ds(off, K)], didx.at[b],
                                 sdi[b])
        return carry

    lax.fori_loop(0, NCH // 2, body2, 0)
    pltpu.sync_copy(src_hbm.at[pl.ds(base + NCH * K, KT)], tsidx)
    pltpu.sync_copy(dst_hbm.at[pl.ds(base + NCH * K, KT)], tdidx)
    pltpu.async_copy(hs_hbm.at[tsidx], trows, sg0).wait()
    pltpu.sync_copy(trows, sp_agg.at[tdidx], add=True)
    plsc.subcore_barrier()
    pltpu.sync_copy(sp_agg.at[pl.ds(s * RPT, RPT)],
                    out_hbm.at[c, pl.ds(s * RPT, RPT)])

    @pl.when(s == NS - 1)
    def _out_tail():
        pltpu.sync_copy(sp_agg.at[pl.ds(NS * RPT, TAIL)],
                        out_hbm.at[c, pl.ds(NS * RPT, TAIL)])


def _dinv_block(d0, d1):
    return lax.rsqrt(d0[:, :1] + d1[:, :1] - 1.0)


def _tc_first_body(x_ref, w_ref, d0_ref, d1_ref, o_ref):
    dv = _dinv_block(d0_ref[...], d1_ref[...])
    h = jnp.dot(x_ref[...], w_ref[...], preferred_element_type=jnp.float32)
    o_ref[...] = h * dv


def _tc_first(x, W1, degf):
    return pl.pallas_call(
        _tc_first_body,
        grid=(NBLK,),
        in_specs=[
            pl.BlockSpec((RB, F), lambda i: (i, 0)),
            pl.BlockSpec((F, F), lambda i: (0, 0)),
            pl.BlockSpec((RB, 8), lambda i: (i, 0)),
            pl.BlockSpec((RB, 8), lambda i: (i + NBLK, 0)),
        ],
        out_specs=pl.BlockSpec((RB, F), lambda i: (i, 0)),
        out_shape=jax.ShapeDtypeStruct((N, F), jnp.float32),
    )(x, W1, degf, degf)


def _tc_mid_body(a0_ref, a1_ref, hs_ref, d0_ref, d1_ref, b_ref, w_ref, o_ref):
    dv = _dinv_block(d0_ref[...], d1_ref[...])
    hs = hs_ref[...]
    z = jax.nn.relu(dv * (a0_ref[...] + a1_ref[...] - hs) + b_ref[...])
    o_ref[...] = jnp.dot(z, w_ref[...], preferred_element_type=jnp.float32) * dv


def _tc_mid(aggf, hs, degf, b, W):
    return pl.pallas_call(
        _tc_mid_body,
        grid=(NBLK,),
        in_specs=[
            pl.BlockSpec((RB, F), lambda i: (i, 0)),
            pl.BlockSpec((RB, F), lambda i: (i + NBLK, 0)),
            pl.BlockSpec((RB, F), lambda i: (i, 0)),
            pl.BlockSpec((RB, 8), lambda i: (i, 0)),
            pl.BlockSpec((RB, 8), lambda i: (i + NBLK, 0)),
            pl.BlockSpec((1, F), lambda i: (0, 0)),
            pl.BlockSpec((F, F), lambda i: (0, 0)),
        ],
        out_specs=pl.BlockSpec((RB, F), lambda i: (i, 0)),
        out_shape=jax.ShapeDtypeStruct((N, F), jnp.float32),
    )(aggf, aggf, hs, degf, degf, b, W)


def _tc_final_body(a0_ref, a1_ref, hs_ref, d0_ref, d1_ref, b_ref, bt_ref,
                   lw_ref, lb_ref, o_ref, acc, cnt):
    i = pl.program_id(0)

    @pl.when(i == 0)
    def _init():
        acc[...] = jnp.zeros_like(acc)
        cnt[...] = jnp.zeros_like(cnt)

    dv = _dinv_block(d0_ref[...], d1_ref[...])
    hs = hs_ref[...]
    z = jax.nn.relu(dv * (a0_ref[...] + a1_ref[...] - hs) + b_ref[...])
    bt = bt_ref[...].reshape(1, RB)
    oh = (lax.broadcasted_iota(jnp.int32, (G, RB), 0) == bt).astype(jnp.float32)
    acc[...] += jnp.dot(oh, z, preferred_element_type=jnp.float32)
    cnt[...] += jnp.broadcast_to(jnp.sum(oh, axis=1, keepdims=True), (G, F))

    @pl.when(i == NBLK - 1)
    def _fin():
        pooled = acc[...] / jnp.maximum(cnt[...], 1.0)
        o_ref[...] = jnp.dot(pooled, lw_ref[...],
                             preferred_element_type=jnp.float32) + lb_ref[...]


def _tc_final(aggf, hs, degf, b, batch2d, linW, linb2d):
    C = linW.shape[1]
    return pl.pallas_call(
        _tc_final_body,
        grid=(NBLK,),
        in_specs=[
            pl.BlockSpec((RB, F), lambda i: (i, 0)),
            pl.BlockSpec((RB, F), lambda i: (i + NBLK, 0)),
            pl.BlockSpec((RB, F), lambda i: (i, 0)),
            pl.BlockSpec((RB, 8), lambda i: (i, 0)),
            pl.BlockSpec((RB, 8), lambda i: (i + NBLK, 0)),
            pl.BlockSpec((1, F), lambda i: (0, 0)),
            pl.BlockSpec((RB, 1), lambda i: (i, 0)),
            pl.BlockSpec((F, C), lambda i: (0, 0)),
            pl.BlockSpec((1, C), lambda i: (0, 0)),
        ],
        out_specs=pl.BlockSpec((G, C), lambda i: (0, 0)),
        out_shape=jax.ShapeDtypeStruct((G, C), jnp.float32),
        scratch_shapes=[
            pltpu.VMEM((G, F), jnp.float32),
            pltpu.VMEM((G, F), jnp.float32),
        ],
    )(aggf, aggf, hs, degf, degf, b, batch2d, linW, linb2d)


def kernel(x, edge_index, batch, W1, b1, W2, b2, linW, linb):
    src = edge_index[0]
    dst = edge_index[1]
    ones = jnp.ones((N, 8), jnp.float32)

    degf = _sc_degree(dst, ones).reshape(NC * N, 8)
    hs1 = _tc_first(x, W1, degf)
    agg1 = _sc_aggregate(hs1, src, dst).reshape(NC * N, F)
    hs2 = _tc_mid(agg1, hs1, degf, b1.reshape(1, F), W2)
    agg2 = _sc_aggregate(hs2, src, dst).reshape(NC * N, F)
    return _tc_final(agg2, hs2, degf, b2.reshape(1, F),
                     batch.reshape(N, 1), linW, linb.reshape(1, linb.shape[0]))


# speedup vs baseline: 28.0081x; 1.0215x over previous
import functools

import jax
import jax.numpy as jnp
from jax import lax
from jax.experimental import pallas as pl
from jax.experimental.pallas import tpu as pltpu
from jax.experimental.pallas import tpu_sc as plsc

N = 10000
E = 320000
F = 128
G = 16
NC = 2
NS = 16
NW = NC * NS
EP = E // NW
K = 128
NCH = EP // K
NCHM = NCH - 2
KT = EP - NCH * K
RPT = 624
TAIL = N - NS * RPT
RB = 2000
NBLK = N // RB

_mesh = plsc.VectorSubcoreMesh(
    core_axis_name="c", subcore_axis_name="s", num_cores=NC, num_subcores=NS)


@functools.partial(
    pl.kernel,
    out_type=jax.ShapeDtypeStruct((NC, N, 8), jnp.float32),
    mesh=_mesh,
    scratch_types=[
        pltpu.VMEM((4, K), jnp.int32),
        pltpu.VMEM((K, 8), jnp.float32),
        pltpu.VMEM((KT,), jnp.int32),
        pltpu.VMEM_SHARED((N, 8), jnp.float32),
        pltpu.SemaphoreType.DMA,
        pltpu.SemaphoreType.DMA,
        pltpu.SemaphoreType.DMA,
        pltpu.SemaphoreType.DMA,
        pltpu.SemaphoreType.DMA,
        pltpu.SemaphoreType.DMA,
    ],
)
def _sc_degree(dst_hbm, ones_hbm, out_hbm, didx, onesrow, tdidx, sp_deg,
               sdi0, sdi1, sdi2, sdi3, ssc0, ssc1):
    c = lax.axis_index("c")
    s = lax.axis_index("s")
    sdi = (sdi0, sdi1, sdi2, sdi3)
    ssc = (ssc0, ssc1)
    pltpu.sync_copy(ones_hbm.at[pl.ds(s * RPT, RPT)],
                    sp_deg.at[pl.ds(s * RPT, RPT)])

    @pl.when(s == NS - 1)
    def _init_tail():
        pltpu.sync_copy(ones_hbm.at[pl.ds(NS * RPT, TAIL)],
                        sp_deg.at[pl.ds(NS * RPT, TAIL)])

    pltpu.sync_copy(ones_hbm.at[pl.ds(0, K)], onesrow)
    plsc.subcore_barrier()
    base = (c * (E // NC) + s * EP)

    def stage(j, q):
        pltpu.async_copy(dst_hbm.at[pl.ds(base + j * K, K)], didx.at[q],
                         sdi[q])

    def wait_stage(q):
        pltpu.make_async_copy(dst_hbm.at[pl.ds(base, K)], didx.at[q],
                              sdi[q]).wait()

    def scat(q, e):
        pltpu.async_copy(onesrow, sp_deg.at[didx.at[q]], ssc[e], add=True)

    def wait_scat(q, e):
        pltpu.make_async_copy(onesrow, sp_deg.at[didx.at[q]], ssc[e]).wait()

    for q in range(3):
        stage(q, q)

    def body4(g, carry):
        for b in range(4):
            j = 4 * g + b
            wait_stage(b)
            scat(b, b % 2)
            if b == 0:
                @pl.when(j >= 1)
                def _drain_prev():
                    wait_scat(3, 1)
            else:
                wait_scat(b - 1, (b - 1) % 2)
            if b == 3:
                @pl.when(j + 3 < NCH)
                def _stage_next():
                    stage(j + 3, 2)
            else:
                stage(j + 3, (b + 3) % 4)
        return carry

    lax.fori_loop(0, NCHM // 4, body4, 0)
    wait_stage(0)
    scat(0, 0)
    wait_scat(3, 1)
    wait_stage(1)
    scat(1, 1)
    wait_scat(0, 0)
    wait_scat(1, 1)
    pltpu.sync_copy(dst_hbm.at[pl.ds(base + NCH * K, KT)], tdidx)
    pltpu.sync_copy(onesrow.at[pl.ds(0, KT)], sp_deg.at[tdidx], add=True)
    plsc.subcore_barrier()
    pltpu.sync_copy(sp_deg.at[pl.ds(s * RPT, RPT)],
                    out_hbm.at[c, pl.ds(s * RPT, RPT)])

    @pl.when(s == NS - 1)
    def _out_tail():
        pltpu.sync_copy(sp_deg.at[pl.ds(NS * RPT, TAIL)],
                        out_hbm.at[c, pl.ds(NS * RPT, TAIL)])


@functools.partial(
    pl.kernel,
    out_type=jax.ShapeDtypeStruct((NC, N, F), jnp.float32),
    mesh=_mesh,
    scratch_types=[
        pltpu.VMEM((2, K), jnp.int32),
        pltpu.VMEM((4, K), jnp.int32),
        pltpu.VMEM((2, K, F), jnp.float32),
        pltpu.VMEM((KT,), jnp.int32),
        pltpu.VMEM((KT,), jnp.int32),
        pltpu.VMEM((KT, F), jnp.float32),
        pltpu.VMEM_SHARED((N, F), jnp.float32),
        pltpu.SemaphoreType.DMA,
        pltpu.SemaphoreType.DMA,
        pltpu.SemaphoreType.DMA,
        pltpu.SemaphoreType.DMA,
        pltpu.SemaphoreType.DMA,
        pltpu.SemaphoreType.DMA,
        pltpu.SemaphoreType.DMA,
        pltpu.SemaphoreType.DMA,
        pltpu.SemaphoreType.DMA,
        pltpu.SemaphoreType.DMA,
    ],
)
def _sc_aggregate(hs_hbm, src_hbm, dst_hbm, out_hbm, sidx, didx, rows,
                  tsidx, tdidx, trows, sp_agg,
                  ssi0, ssi1, sdi0, sdi1, sdi2, sdi3, sg0, sg1, ss0, ss1):
    c = lax.axis_index("c")
    s = lax.axis_index("s")
    ssi = (ssi0, ssi1)
    sdi = (sdi0, sdi1, sdi2, sdi3)
    sg = (sg0, sg1)
    ss = (ss0, ss1)
    pltpu.sync_copy(hs_hbm.at[pl.ds(s * RPT, RPT)],
                    sp_agg.at[pl.ds(s * RPT, RPT)])

    @pl.when(s == NS - 1)
    def _init_tail():
        pltpu.sync_copy(hs_hbm.at[pl.ds(NS * RPT, TAIL)],
                        sp_agg.at[pl.ds(NS * RPT, TAIL)])

    plsc.subcore_barrier()
    base = (c * (E // NC) + s * EP)

    def stage_s(j, p):
        pltpu.async_copy(src_hbm.at[pl.ds(base + j * K, K)], sidx.at[p],
                         ssi[p])

    def wait_s(p):
        pltpu.make_async_copy(src_hbm.at[pl.ds(base, K)], sidx.at[p],
                              ssi[p]).wait()

    def stage_d(j, q):
        pltpu.async_copy(dst_hbm.at[pl.ds(base + j * K, K)], didx.at[q],
                         sdi[q])

    def wait_d(q):
        pltpu.make_async_copy(dst_hbm.at[pl.ds(base, K)], didx.at[q],
                              sdi[q]).wait()

    def gat(p):
        pltpu.async_copy(hs_hbm.at[sidx.at[p]], rows.at[p], sg[p])

    def wait_g(p):
        pltpu.make_async_copy(hs_hbm.at[sidx.at[p]], rows.at[p],
                              sg[p]).wait()

    def scat(p, q):
        pltpu.async_copy(rows.at[p], sp_agg.at[didx.at[q]], ss[p], add=True)

    def wait_sc(p):
        pltpu.make_async_copy(rows.at[p], sp_agg.at[didx.at[0]],
                              ss[p]).wait()

    stage_s(0, 0)
    stage_s(1, 1)
    for q in range(3):
        stage_d(q, q)
    wait_s(0)
    gat(0)

    def body4(g, carry):
        for b in range(4):
            j = 4 * g + b
            r = b % 2
            orr = 1 - r
            wait_g(r)
            stage_s(j + 2, r)
            wait_d(b)
            scat(r, b)
            wait_s(orr)
            if b == 0:
                @pl.when(j >= 1)
                def _drain_prev():
                    wait_sc(orr)
            else:
                wait_sc(orr)
            gat(orr)
            if b == 3:
                @pl.when(j + 3 < NCH)
                def _stage_next():
                    stage_d(j + 3, 2)
            else:
                stage_d(j + 3, (b + 3) % 4)
        return carry

    lax.fori_loop(0, NCHM // 4, body4, 0)
    wait_g(0)
    wait_d(0)
    scat(0, 0)
    wait_s(1)
    wait_sc(1)
    gat(1)
    wait_g(1)
    wait_d(1)
    scat(1, 1)
    wait_sc(0)
    wait_sc(1)
    pltpu.sync_copy(src_hbm.at[pl.ds(base + NCH * K, KT)], tsidx)
    pltpu.sync_copy(dst_hbm.at[pl.ds(base + NCH * K, KT)], tdidx)
    pltpu.async_copy(hs_hbm.at[tsidx], trows, sg0).wait()
    pltpu.sync_copy(trows, sp_agg.at[tdidx], add=True)
    plsc.subcore_barrier()
    pltpu.sync_copy(sp_agg.at[pl.ds(s * RPT, RPT)],
                    out_hbm.at[c, pl.ds(s * RPT, RPT)])

    @pl.when(s == NS - 1)
    def _out_tail():
        pltpu.sync_copy(sp_agg.at[pl.ds(NS * RPT, TAIL)],
                        out_hbm.at[c, pl.ds(NS * RPT, TAIL)])


def _dinv_block(d0, d1):
    return lax.rsqrt(d0[:, :1] + d1[:, :1] - 1.0)


def _tc_first_body(x_ref, w_ref, d0_ref, d1_ref, o_ref):
    dv = _dinv_block(d0_ref[...], d1_ref[...])
    h = jnp.dot(x_ref[...], w_ref[...], preferred_element_type=jnp.float32)
    o_ref[...] = h * dv


def _tc_first(x, W1, degf):
    return pl.pallas_call(
        _tc_first_body,
        grid=(NBLK,),
        in_specs=[
            pl.BlockSpec((RB, F), lambda i: (i, 0)),
            pl.BlockSpec((F, F), lambda i: (0, 0)),
            pl.BlockSpec((RB, 8), lambda i: (i, 0)),
            pl.BlockSpec((RB, 8), lambda i: (i + NBLK, 0)),
        ],
        out_specs=pl.BlockSpec((RB, F), lambda i: (i, 0)),
        out_shape=jax.ShapeDtypeStruct((N, F), jnp.float32),
    )(x, W1, degf, degf)


def _tc_mid_body(a0_ref, a1_ref, hs_ref, d0_ref, d1_ref, b_ref, w_ref, o_ref):
    dv = _dinv_block(d0_ref[...], d1_ref[...])
    hs = hs_ref[...]
    z = jax.nn.relu(dv * (a0_ref[...] + a1_ref[...] - hs) + b_ref[...])
    o_ref[...] = jnp.dot(z, w_ref[...], preferred_element_type=jnp.float32) * dv


def _tc_mid(aggf, hs, degf, b, W):
    return pl.pallas_call(
        _tc_mid_body,
        grid=(NBLK,),
        in_specs=[
            pl.BlockSpec((RB, F), lambda i: (i, 0)),
            pl.BlockSpec((RB, F), lambda i: (i + NBLK, 0)),
            pl.BlockSpec((RB, F), lambda i: (i, 0)),
            pl.BlockSpec((RB, 8), lambda i: (i, 0)),
            pl.BlockSpec((RB, 8), lambda i: (i + NBLK, 0)),
            pl.BlockSpec((1, F), lambda i: (0, 0)),
            pl.BlockSpec((F, F), lambda i: (0, 0)),
        ],
        out_specs=pl.BlockSpec((RB, F), lambda i: (i, 0)),
        out_shape=jax.ShapeDtypeStruct((N, F), jnp.float32),
    )(aggf, aggf, hs, degf, degf, b, W)


def _tc_final_body(a0_ref, a1_ref, hs_ref, d0_ref, d1_ref, b_ref, bt_ref,
                   lw_ref, lb_ref, o_ref, acc, cnt):
    i = pl.program_id(0)

    @pl.when(i == 0)
    def _init():
        acc[...] = jnp.zeros_like(acc)
        cnt[...] = jnp.zeros_like(cnt)

    dv = _dinv_block(d0_ref[...], d1_ref[...])
    hs = hs_ref[...]
    z = jax.nn.relu(dv * (a0_ref[...] + a1_ref[...] - hs) + b_ref[...])
    bt = bt_ref[...].reshape(1, RB)
    oh = (lax.broadcasted_iota(jnp.int32, (G, RB), 0) == bt).astype(jnp.float32)
    acc[...] += jnp.dot(oh, z, preferred_element_type=jnp.float32)
    cnt[...] += jnp.broadcast_to(jnp.sum(oh, axis=1, keepdims=True), (G, F))

    @pl.when(i == NBLK - 1)
    def _fin():
        pooled = acc[...] / jnp.maximum(cnt[...], 1.0)
        o_ref[...] = jnp.dot(pooled, lw_ref[...],
                             preferred_element_type=jnp.float32) + lb_ref[...]


def _tc_final(aggf, hs, degf, b, batch2d, linW, linb2d):
    C = linW.shape[1]
    return pl.pallas_call(
        _tc_final_body,
        grid=(NBLK,),
        in_specs=[
            pl.BlockSpec((RB, F), lambda i: (i, 0)),
            pl.BlockSpec((RB, F), lambda i: (i + NBLK, 0)),
            pl.BlockSpec((RB, F), lambda i: (i, 0)),
            pl.BlockSpec((RB, 8), lambda i: (i, 0)),
            pl.BlockSpec((RB, 8), lambda i: (i + NBLK, 0)),
            pl.BlockSpec((1, F), lambda i: (0, 0)),
            pl.BlockSpec((RB, 1), lambda i: (i, 0)),
            pl.BlockSpec((F, C), lambda i: (0, 0)),
            pl.BlockSpec((1, C), lambda i: (0, 0)),
        ],
        out_specs=pl.BlockSpec((G, C), lambda i: (0, 0)),
        out_shape=jax.ShapeDtypeStruct((G, C), jnp.float32),
        scratch_shapes=[
            pltpu.VMEM((G, F), jnp.float32),
            pltpu.VMEM((G, F), jnp.float32),
        ],
    )(aggf, aggf, hs, degf, degf, b, batch2d, linW, linb2d)


def kernel(x, edge_index, batch, W1, b1, W2, b2, linW, linb):
    src = edge_index[0]
    dst = edge_index[1]
    ones = jnp.ones((N, 8), jnp.float32)

    degf = _sc_degree(dst, ones).reshape(NC * N, 8)
    hs1 = _tc_first(x, W1, degf)
    agg1 = _sc_aggregate(hs1, src, dst).reshape(NC * N, F)
    hs2 = _tc_mid(agg1, hs1, degf, b1.reshape(1, F), W2)
    agg2 = _sc_aggregate(hs2, src, dst).reshape(NC * N, F)
    return _tc_final(agg2, hs2, degf, b2.reshape(1, F),
                     batch.reshape(N, 1), linW, linb.reshape(1, linb.shape[0]))


# speedup vs baseline: 34.1405x; 1.2190x over previous
import functools

import jax
import jax.numpy as jnp
from jax import lax
from jax.experimental import pallas as pl
from jax.experimental.pallas import tpu as pltpu
from jax.experimental.pallas import tpu_sc as plsc

N = 10000
E = 320000
F = 128
G = 16
NC = 2
NS = 16
NW = NC * NS
EP = E // NW
K = 128
NCH = EP // K
NCHM = NCH - 2
KT = EP - NCH * K
KA = 80
NCHA = EP // KA
NCHAM = NCHA - 1
RPT = 624
TAIL = N - NS * RPT
RB = 2000
NBLK = N // RB

_mesh = plsc.VectorSubcoreMesh(
    core_axis_name="c", subcore_axis_name="s", num_cores=NC, num_subcores=NS)


@functools.partial(
    pl.kernel,
    out_type=jax.ShapeDtypeStruct((NC, N, 8), jnp.float32),
    mesh=_mesh,
    scratch_types=[
        pltpu.VMEM((4, K), jnp.int32),
        pltpu.VMEM((K, 8), jnp.float32),
        pltpu.VMEM((KT,), jnp.int32),
        pltpu.VMEM_SHARED((N, 8), jnp.float32),
        pltpu.SemaphoreType.DMA,
        pltpu.SemaphoreType.DMA,
        pltpu.SemaphoreType.DMA,
        pltpu.SemaphoreType.DMA,
        pltpu.SemaphoreType.DMA,
        pltpu.SemaphoreType.DMA,
    ],
)
def _sc_degree(dst_hbm, ones_hbm, out_hbm, didx, onesrow, tdidx, sp_deg,
               sdi0, sdi1, sdi2, sdi3, ssc0, ssc1):
    c = lax.axis_index("c")
    s = lax.axis_index("s")
    sdi = (sdi0, sdi1, sdi2, sdi3)
    ssc = (ssc0, ssc1)
    pltpu.sync_copy(ones_hbm.at[pl.ds(s * RPT, RPT)],
                    sp_deg.at[pl.ds(s * RPT, RPT)])

    @pl.when(s == NS - 1)
    def _init_tail():
        pltpu.sync_copy(ones_hbm.at[pl.ds(NS * RPT, TAIL)],
                        sp_deg.at[pl.ds(NS * RPT, TAIL)])

    pltpu.sync_copy(ones_hbm.at[pl.ds(0, K)], onesrow)
    plsc.subcore_barrier()
    base = (c * (E // NC) + s * EP)

    def stage(j, q):
        pltpu.async_copy(dst_hbm.at[pl.ds(base + j * K, K)], didx.at[q],
                         sdi[q])

    def wait_stage(q):
        pltpu.make_async_copy(dst_hbm.at[pl.ds(base, K)], didx.at[q],
                              sdi[q]).wait()

    def scat(q, e):
        pltpu.async_copy(onesrow, sp_deg.at[didx.at[q]], ssc[e], add=True)

    def wait_scat(q, e):
        pltpu.make_async_copy(onesrow, sp_deg.at[didx.at[q]], ssc[e]).wait()

    for q in range(3):
        stage(q, q)

    def body4(g, carry):
        for b in range(4):
            j = 4 * g + b
            wait_stage(b)
            scat(b, b % 2)
            if b == 0:
                @pl.when(j >= 1)
                def _drain_prev():
                    wait_scat(3, 1)
            else:
                wait_scat(b - 1, (b - 1) % 2)
            if b == 3:
                @pl.when(j + 3 < NCH)
                def _stage_next():
                    stage(j + 3, 2)
            else:
                stage(j + 3, (b + 3) % 4)
        return carry

    lax.fori_loop(0, NCHM // 4, body4, 0)
    wait_stage(0)
    scat(0, 0)
    wait_scat(3, 1)
    wait_stage(1)
    scat(1, 1)
    wait_scat(0, 0)
    wait_scat(1, 1)
    pltpu.sync_copy(dst_hbm.at[pl.ds(base + NCH * K, KT)], tdidx)
    pltpu.sync_copy(onesrow.at[pl.ds(0, KT)], sp_deg.at[tdidx], add=True)
    plsc.subcore_barrier()
    pltpu.sync_copy(sp_deg.at[pl.ds(s * RPT, RPT)],
                    out_hbm.at[c, pl.ds(s * RPT, RPT)])

    @pl.when(s == NS - 1)
    def _out_tail():
        pltpu.sync_copy(sp_deg.at[pl.ds(NS * RPT, TAIL)],
                        out_hbm.at[c, pl.ds(NS * RPT, TAIL)])


@functools.partial(
    pl.kernel,
    out_type=jax.ShapeDtypeStruct((NC, N, F), jnp.float32),
    mesh=_mesh,
    scratch_types=[
        pltpu.VMEM((4, KA), jnp.int32),
        pltpu.VMEM((4, KA), jnp.int32),
        pltpu.VMEM((4, KA, F), jnp.float32),
        pltpu.VMEM_SHARED((N, F), jnp.float32),
        pltpu.SemaphoreType.DMA,
        pltpu.SemaphoreType.DMA,
        pltpu.SemaphoreType.DMA,
        pltpu.SemaphoreType.DMA,
        pltpu.SemaphoreType.DMA,
        pltpu.SemaphoreType.DMA,
        pltpu.SemaphoreType.DMA,
        pltpu.SemaphoreType.DMA,
        pltpu.SemaphoreType.DMA,
        pltpu.SemaphoreType.DMA,
        pltpu.SemaphoreType.DMA,
        pltpu.SemaphoreType.DMA,
        pltpu.SemaphoreType.DMA,
        pltpu.SemaphoreType.DMA,
    ],
)
def _sc_aggregate(hs_hbm, src_hbm, dst_hbm, out_hbm, sidx, didx, rows,
                  sp_agg,
                  ssi0, ssi1, ssi2, ssi3, sdi0, sdi1, sdi2, sdi3,
                  sg0, sg1, sg2, sg3, ss0, ss1):
    c = lax.axis_index("c")
    s = lax.axis_index("s")
    ssi = (ssi0, ssi1, ssi2, ssi3)
    sdi = (sdi0, sdi1, sdi2, sdi3)
    sg = (sg0, sg1, sg2, sg3)
    ss = (ss0, ss1)
    pltpu.sync_copy(hs_hbm.at[pl.ds(s * RPT, RPT)],
                    sp_agg.at[pl.ds(s * RPT, RPT)])

    @pl.when(s == NS - 1)
    def _init_tail():
        pltpu.sync_copy(hs_hbm.at[pl.ds(NS * RPT, TAIL)],
                        sp_agg.at[pl.ds(NS * RPT, TAIL)])

    plsc.subcore_barrier()
    base = (c * (E // NC) + s * EP)

    def stage_s(j, p):
        pltpu.async_copy(src_hbm.at[pl.ds(base + j * KA, KA)], sidx.at[p],
                         ssi[p])

    def wait_s(p):
        pltpu.make_async_copy(src_hbm.at[pl.ds(base, KA)], sidx.at[p],
                              ssi[p]).wait()

    def stage_d(j, q):
        pltpu.async_copy(dst_hbm.at[pl.ds(base + j * KA, KA)], didx.at[q],
                         sdi[q])

    def wait_d(q):
        pltpu.make_async_copy(dst_hbm.at[pl.ds(base, KA)], didx.at[q],
                              sdi[q]).wait()

    def gat(p):
        pltpu.async_copy(hs_hbm.at[sidx.at[p]], rows.at[p], sg[p])

    def wait_g(p):
        pltpu.make_async_copy(hs_hbm.at[sidx.at[p]], rows.at[p],
                              sg[p]).wait()

    def scat(p, q, e):
        pltpu.async_copy(rows.at[p], sp_agg.at[didx.at[q]], ss[e], add=True)

    def wait_sc(e):
        pltpu.make_async_copy(rows.at[0], sp_agg.at[didx.at[0]],
                              ss[e]).wait()

    for q in range(4):
        stage_s(q, q)
    for q in range(3):
        stage_d(q, q)
    for q in range(3):
        wait_s(q)
        gat(q)

    def body4(g, carry):
        for b in range(4):
            j = 4 * g + b
            bp = (b + 3) % 4
            e = b % 2
            oe = 1 - e
            wait_g(b)
            wait_d(b)
            scat(b, b, e)
            if b == 0:
                @pl.when(j >= 1)
                def _drain_prev():
                    wait_sc(oe)
            else:
                wait_sc(oe)
            if b >= 1:
                @pl.when(j + 4 < NCHA)
                def _stage_s_next():
                    stage_s(j + 4, b)
            else:
                stage_s(j + 4, b)
            if b >= 2:
                @pl.when(j + 3 < NCHA)
                def _next_gather():
                    stage_d(j + 3, bp)
                    wait_s(bp)
                    gat(bp)
            else:
                stage_d(j + 3, bp)
                wait_s(bp)
                gat(bp)
        return carry

    lax.fori_loop(0, NCHAM // 4, body4, 0)
    wait_g(0)
    wait_d(0)
    scat(0, 0, 0)
    wait_sc(1)
    wait_sc(0)
    plsc.subcore_barrier()
    pltpu.sync_copy(sp_agg.at[pl.ds(s * RPT, RPT)],
                    out_hbm.at[c, pl.ds(s * RPT, RPT)])

    @pl.when(s == NS - 1)
    def _out_tail():
        pltpu.sync_copy(sp_agg.at[pl.ds(NS * RPT, TAIL)],
                        out_hbm.at[c, pl.ds(NS * RPT, TAIL)])


def _dinv_block(d0, d1):
    return lax.rsqrt(d0[:, :1] + d1[:, :1] - 1.0)


def _tc_first_body(x_ref, w_ref, d0_ref, d1_ref, o_ref):
    dv = _dinv_block(d0_ref[...], d1_ref[...])
    h = jnp.dot(x_ref[...], w_ref[...], preferred_element_type=jnp.float32)
    o_ref[...] = h * dv


def _tc_first(x, W1, degf):
    return pl.pallas_call(
        _tc_first_body,
        grid=(NBLK,),
        in_specs=[
            pl.BlockSpec((RB, F), lambda i: (i, 0)),
            pl.BlockSpec((F, F), lambda i: (0, 0)),
            pl.BlockSpec((RB, 8), lambda i: (i, 0)),
            pl.BlockSpec((RB, 8), lambda i: (i + NBLK, 0)),
        ],
        out_specs=pl.BlockSpec((RB, F), lambda i: (i, 0)),
        out_shape=jax.ShapeDtypeStruct((N, F), jnp.float32),
    )(x, W1, degf, degf)


def _tc_mid_body(a0_ref, a1_ref, hs_ref, d0_ref, d1_ref, b_ref, w_ref, o_ref):
    dv = _dinv_block(d0_ref[...], d1_ref[...])
    hs = hs_ref[...]
    z = jax.nn.relu(dv * (a0_ref[...] + a1_ref[...] - hs) + b_ref[...])
    o_ref[...] = jnp.dot(z, w_ref[...], preferred_element_type=jnp.float32) * dv


def _tc_mid(aggf, hs, degf, b, W):
    return pl.pallas_call(
        _tc_mid_body,
        grid=(NBLK,),
        in_specs=[
            pl.BlockSpec((RB, F), lambda i: (i, 0)),
            pl.BlockSpec((RB, F), lambda i: (i + NBLK, 0)),
            pl.BlockSpec((RB, F), lambda i: (i, 0)),
            pl.BlockSpec((RB, 8), lambda i: (i, 0)),
            pl.BlockSpec((RB, 8), lambda i: (i + NBLK, 0)),
            pl.BlockSpec((1, F), lambda i: (0, 0)),
            pl.BlockSpec((F, F), lambda i: (0, 0)),
        ],
        out_specs=pl.BlockSpec((RB, F), lambda i: (i, 0)),
        out_shape=jax.ShapeDtypeStruct((N, F), jnp.float32),
    )(aggf, aggf, hs, degf, degf, b, W)


def _tc_final_body(a0_ref, a1_ref, hs_ref, d0_ref, d1_ref, b_ref, bt_ref,
                   lw_ref, lb_ref, o_ref, acc, cnt):
    i = pl.program_id(0)

    @pl.when(i == 0)
    def _init():
        acc[...] = jnp.zeros_like(acc)
        cnt[...] = jnp.zeros_like(cnt)

    dv = _dinv_block(d0_ref[...], d1_ref[...])
    hs = hs_ref[...]
    z = jax.nn.relu(dv * (a0_ref[...] + a1_ref[...] - hs) + b_ref[...])
    bt = bt_ref[...].reshape(1, RB)
    oh = (lax.broadcasted_iota(jnp.int32, (G, RB), 0) == bt).astype(jnp.float32)
    acc[...] += jnp.dot(oh, z, preferred_element_type=jnp.float32)
    cnt[...] += jnp.broadcast_to(jnp.sum(oh, axis=1, keepdims=True), (G, F))

    @pl.when(i == NBLK - 1)
    def _fin():
        pooled = acc[...] / jnp.maximum(cnt[...], 1.0)
        o_ref[...] = jnp.dot(pooled, lw_ref[...],
                             preferred_element_type=jnp.float32) + lb_ref[...]


def _tc_final(aggf, hs, degf, b, batch2d, linW, linb2d):
    C = linW.shape[1]
    return pl.pallas_call(
        _tc_final_body,
        grid=(NBLK,),
        in_specs=[
            pl.BlockSpec((RB, F), lambda i: (i, 0)),
            pl.BlockSpec((RB, F), lambda i: (i + NBLK, 0)),
            pl.BlockSpec((RB, F), lambda i: (i, 0)),
            pl.BlockSpec((RB, 8), lambda i: (i, 0)),
            pl.BlockSpec((RB, 8), lambda i: (i + NBLK, 0)),
            pl.BlockSpec((1, F), lambda i: (0, 0)),
            pl.BlockSpec((RB, 1), lambda i: (i, 0)),
            pl.BlockSpec((F, C), lambda i: (0, 0)),
            pl.BlockSpec((1, C), lambda i: (0, 0)),
        ],
        out_specs=pl.BlockSpec((G, C), lambda i: (0, 0)),
        out_shape=jax.ShapeDtypeStruct((G, C), jnp.float32),
        scratch_shapes=[
            pltpu.VMEM((G, F), jnp.float32),
            pltpu.VMEM((G, F), jnp.float32),
        ],
    )(aggf, aggf, hs, degf, degf, b, batch2d, linW, linb2d)


def kernel(x, edge_index, batch, W1, b1, W2, b2, linW, linb):
    src = edge_index[0]
    dst = edge_index[1]
    ones = jnp.ones((N, 8), jnp.float32)

    degf = _sc_degree(dst, ones).reshape(NC * N, 8)
    hs1 = _tc_first(x, W1, degf)
    agg1 = _sc_aggregate(hs1, src, dst).reshape(NC * N, F)
    hs2 = _tc_mid(agg1, hs1, degf, b1.reshape(1, F), W2)
    agg2 = _sc_aggregate(hs2, src, dst).reshape(NC * N, F)
    return _tc_final(agg2, hs2, degf, b2.reshape(1, F),
                     batch.reshape(N, 1), linW, linb.reshape(1, linb.shape[0]))
